# Initial kernel scaffold; baseline (speedup 1.0000x reference)
import jax
import jax.numpy as jnp
from jax.experimental import pallas as pl


def kernel(x, edge_index, batch, W_proj, W1_0, b1_0, W2_0, b2_0, W1_1, b1_1, W2_1, b2_1):
    raise NotImplementedError("write your pallas kernel here")


# speedup vs baseline: 5.7483x; 5.7483x over previous
import functools

import jax
import jax.numpy as jnp
from jax import lax
from jax.experimental import pallas as pl
from jax.experimental.pallas import tpu as pltpu
from jax.experimental.pallas import tpu_sc as plsc

N = 10000
E = 320000
D = 128
G = 128

NPAD = 10240
CH = 128
NCHUNK = E // CH
NW = 32
BASE_ITERS = NCHUNK // NW
EXTRA = NCHUNK - BASE_ITERS * NW

ROWS_BLK = 1000
GRID = N // ROWS_BLK


def _agg_body(h_hbm, src_hbm, dst_hbm, zeros_hbm, out_hbm,
              src_v, dst_v, rows_v, acc_sh, sem):
    cid = lax.axis_index("c")
    sid = lax.axis_index("s")
    wid = sid * 2 + cid

    pltpu.sync_copy(zeros_hbm.at[pl.ds(sid * 640, 640)],
                    acc_sh.at[pl.ds(sid * 640, 640)])
    plsc.subcore_barrier()

    n_iters = BASE_ITERS + jnp.where(wid < EXTRA, 1, 0)

    def body(i, carry):
        base = (i * NW + wid) * CH
        pltpu.sync_copy(src_hbm.at[pl.ds(base, CH)], src_v)
        pltpu.sync_copy(dst_hbm.at[pl.ds(base, CH)], dst_v)
        pltpu.async_copy(h_hbm.at[src_v], rows_v, sem).wait()
        pltpu.sync_copy(rows_v, acc_sh.at[dst_v], add=True)
        return carry

    lax.fori_loop(0, n_iters, body, 0)
    plsc.subcore_barrier()

    wbase = jnp.minimum(sid * 632, N - 632)
    pltpu.sync_copy(acc_sh.at[pl.ds(wbase, 632)],
                    out_hbm.at[cid, pl.ds(wbase, 632)])


_agg_call = pl.kernel(
    _agg_body,
    out_type=jax.ShapeDtypeStruct((2, N, D), jnp.float32),
    mesh=plsc.VectorSubcoreMesh(core_axis_name="c", subcore_axis_name="s"),
    scratch_types=[
        pltpu.VMEM((CH,), jnp.int32),
        pltpu.VMEM((CH,), jnp.int32),
        pltpu.VMEM((CH, D), jnp.float32),
        pltpu.VMEM_SHARED((NPAD, D), jnp.float32),
        pltpu.SemaphoreType.DMA,
    ],
)


def _proj_body(x_ref, w_ref, o_ref):
    o_ref[...] = jnp.dot(x_ref[...], w_ref[...],
                         preferred_element_type=jnp.float32)


def _mlp_body(agg_ref, h_ref, w1_ref, b1_ref, w2_ref, b2_ref, o_ref):
    z = agg_ref[0] + agg_ref[1] + h_ref[...]
    z = jnp.maximum(
        jnp.dot(z, w1_ref[...], preferred_element_type=jnp.float32)
        + b1_ref[...], 0.0)
    z = jnp.maximum(
        jnp.dot(z, w2_ref[...], preferred_element_type=jnp.float32)
        + b2_ref[...], 0.0)
    o_ref[...] = z


def _final_body(agg_ref, h_ref, w1_ref, b1_ref, w2_ref, b2_ref, batch_ref,
                o_ref):
    z = agg_ref[0] + agg_ref[1] + h_ref[...]
    z = jnp.maximum(
        jnp.dot(z, w1_ref[...], preferred_element_type=jnp.float32)
        + b1_ref[...], 0.0)
    z = jnp.maximum(
        jnp.dot(z, w2_ref[...], preferred_element_type=jnp.float32)
        + b2_ref[...], 0.0)
    b = batch_ref[0, 0, :]
    onehot = (b[:, None]
              == lax.broadcasted_iota(jnp.int32, (ROWS_BLK, G), 1)
              ).astype(jnp.float32)
    contrib = lax.dot_general(onehot, z, (((0,), (0,)), ((), ())),
                              preferred_element_type=jnp.float32)

    @pl.when(pl.program_id(0) == 0)
    def _():
        o_ref[...] = jnp.zeros_like(o_ref)

    o_ref[...] += contrib


_W_SPEC = pl.BlockSpec((D, D), lambda i: (0, 0))
_B_SPEC = pl.BlockSpec((1, D), lambda i: (0, 0))
_ROW_SPEC = pl.BlockSpec((ROWS_BLK, D), lambda i: (i, 0))
_AGG_SPEC = pl.BlockSpec((2, ROWS_BLK, D), lambda i: (0, i, 0))

_proj_call = pl.pallas_call(
    _proj_body,
    grid=(GRID,),
    in_specs=[_ROW_SPEC, _W_SPEC],
    out_specs=_ROW_SPEC,
    out_shape=jax.ShapeDtypeStruct((N, D), jnp.float32),
)

_mlp_call = pl.pallas_call(
    _mlp_body,
    grid=(GRID,),
    in_specs=[_AGG_SPEC, _ROW_SPEC, _W_SPEC, _B_SPEC, _W_SPEC, _B_SPEC],
    out_specs=_ROW_SPEC,
    out_shape=jax.ShapeDtypeStruct((N, D), jnp.float32),
)

_final_call = pl.pallas_call(
    _final_body,
    grid=(GRID,),
    in_specs=[_AGG_SPEC, _ROW_SPEC, _W_SPEC, _B_SPEC, _W_SPEC, _B_SPEC,
              pl.BlockSpec((1, 1, ROWS_BLK), lambda i: (i, 0, 0))],
    out_specs=pl.BlockSpec((G, D), lambda i: (0, 0)),
    out_shape=jax.ShapeDtypeStruct((G, D), jnp.float32),
)


def kernel(x, edge_index, batch, W_proj, W1_0, b1_0, W2_0, b2_0,
           W1_1, b1_1, W2_1, b2_1):
    src = edge_index[0]
    dst = edge_index[1]
    zeros = jnp.zeros((NPAD, D), jnp.float32)
    batch3d = batch.reshape(GRID, 1, ROWS_BLK)

    h = _proj_call(x, W_proj)

    agg = _agg_call(h, src, dst, zeros)
    h = _mlp_call(agg, h, W1_0, b1_0.reshape(1, D), W2_0, b2_0.reshape(1, D))

    agg = _agg_call(h, src, dst, zeros)
    out = _final_call(agg, h, W1_1, b1_1.reshape(1, D), W2_1,
                      b2_1.reshape(1, D), batch3d)
    return out
